# Initial kernel scaffold; baseline (speedup 1.0000x reference)
import jax
import jax.numpy as jnp
from jax.experimental import pallas as pl


def kernel(x, edge_index, pestat, batch, W_in, b_in, ln_g, ln_b, W_conv, b_conv, Wa_in1, ba_in1, Wa_in2, ba_in2, Wa_out1, ba_out1, Wa_out2, ba_out2, W_fin, b_fin):
    raise NotImplementedError("write your pallas kernel here")


# speedup vs baseline: 8.7291x; 8.7291x over previous
import functools

import jax
import jax.numpy as jnp
from jax import lax
from jax.experimental import pallas as pl
from jax.experimental.pallas import tpu as pltpu
from jax.experimental.pallas import tpu_sc as plsc

N = 10000
E = 320000
D_FEAT = 128
ENV_DIM = 128
HIDDEN = 64
NUM_LAYERS = 3
HIST_DIM = 12
NUM_GRAPHS = 64
OUT_DIM = 40
TAU = 0.5

_NC, _NS = 2, 16
_NW = _NC * _NS
_C = 128
_NCHUNK = 79
_EPW = _C * _NCHUNK
_EPAD = _EPW * _NW
_DT = 128
_NROW = 10240
_RPT = _NROW // _NS
_HR = _NROW // 128

_R = 512
_NB = _NROW // _R


def _sc_pass_body(table, u, v, kvec, out, out_h,
                  u_idx, v_idx, rows, kv_v, hist, idx_h, acc, acc_h, sem):
    cid = lax.axis_index("c")
    sid = lax.axis_index("s")
    wid = cid * _NS + sid

    zero16 = jnp.zeros((16,), jnp.float32)

    def _zr(i, carry):
        for j in range(_DT // 16):
            rows[i, pl.ds(j * 16, 16)] = zero16
        return carry
    lax.fori_loop(0, _C, _zr, 0)

    def _zh(i, carry):
        for j in range(128 // 16):
            hist[i, pl.ds(j * 16, 16)] = zero16
        return carry
    lax.fori_loop(0, _HR, _zh, 0)

    iota16 = lax.iota(jnp.int32, 16)
    for j in range(_HR // 16):
        idx_h[pl.ds(j * 16, 16)] = iota16 + j * 16

    pltpu.sync_copy(kvec, kv_v)

    for k in range(_RPT // _C):
        pltpu.sync_copy(rows, acc.at[pl.ds(sid * _RPT + k * _C, _C)])

    @pl.when(sid < _HR // 8)
    def _():
        pltpu.sync_copy(rows.at[pl.ds(0, 8)], acc_h.at[pl.ds(sid * 8, 8)])
    plsc.subcore_barrier()

    base = wid * _EPW

    def _chunk(ci, carry):
        off = base + ci * _C
        pltpu.sync_copy(u.at[pl.ds(off, _C)], u_idx)
        pltpu.sync_copy(v.at[pl.ds(off, _C)], v_idx)
        gather = pltpu.async_copy(table.at[u_idx], rows, sem)
        for j in range(_C // 16):
            u16 = u_idx[pl.ds(j * 16, 16)]
            v16 = v_idx[pl.ds(j * 16, 16)]
            kv16 = plsc.load_gather(kv_v, [u16])
            plsc.addupdate_scatter(hist, [v16 >> 7, v16 & 127], kv16)
        gather.wait()
        pltpu.sync_copy(rows, acc.at[v_idx], add=True)
        return carry
    lax.fori_loop(0, _NCHUNK, _chunk, 0)

    pltpu.sync_copy(hist, acc_h.at[idx_h], add=True)
    plsc.subcore_barrier()

    pltpu.sync_copy(acc.at[pl.ds(sid * _RPT, _RPT)],
                    out.at[cid, pl.ds(sid * _RPT, _RPT)])

    @pl.when(sid < _HR // 8)
    def _():
        pltpu.sync_copy(acc_h.at[pl.ds(sid * 8, 8)],
                        out_h.at[cid, pl.ds(sid * 8, 8)])


@functools.cache
def _get_sc_pass():
    return pl.kernel(
        _sc_pass_body,
        out_type=(jax.ShapeDtypeStruct((_NC, _NROW, _DT), jnp.float32),
                  jax.ShapeDtypeStruct((_NC, _HR, 128), jnp.float32)),
        mesh=plsc.VectorSubcoreMesh(core_axis_name="c", subcore_axis_name="s",
                                    num_cores=_NC, num_subcores=_NS),
        compiler_params=pltpu.CompilerParams(needs_layout_passes=False),
        scratch_types=[
            pltpu.VMEM((_C,), jnp.int32),
            pltpu.VMEM((_C,), jnp.int32),
            pltpu.VMEM((_C, _DT), jnp.float32),
            pltpu.VMEM((_NROW,), jnp.float32),
            pltpu.VMEM((_HR, 128), jnp.float32),
            pltpu.VMEM((_HR,), jnp.int32),
            pltpu.VMEM_SHARED((_NROW, _DT), jnp.float32),
            pltpu.VMEM_SHARED((_HR, 128), jnp.float32),
            pltpu.SemaphoreType.DMA,
        ],
    )


def _sc_pass(table, u, v, kvec):
    return _get_sc_pass()(table, u, v, kvec)


def _row_spec(w):
    return pl.BlockSpec((_R, w), lambda i: (i, 0))


def _full_spec(shape):
    nd = len(shape)
    return pl.BlockSpec(shape, lambda i, _n=nd: (0,) * _n)


def _ln(hb, g, b):
    mu = jnp.mean(hb, axis=-1, keepdims=True)
    var = jnp.mean((hb - mu) ** 2, axis=-1, keepdims=True)
    return (hb - mu) / jnp.sqrt(var + 1e-5) * g + b


def _enc_body(x_ref, w_ref, b_ref, h_ref):
    h_ref[...] = jax.nn.relu(jnp.dot(x_ref[...], w_ref[...]) + b_ref[...])


def _enc(x, W_in, b_in):
    return pl.pallas_call(
        _enc_body,
        grid=(_NB,),
        in_specs=[_row_spec(D_FEAT), _full_spec((D_FEAT, ENV_DIM)),
                  _full_spec((1, ENV_DIM))],
        out_specs=_row_spec(ENV_DIM),
        out_shape=jax.ShapeDtypeStruct((N, ENV_DIM), jnp.float32),
    )(x, W_in, b_in.reshape(1, ENV_DIM))


def _build_a_body(h_ref, hist_ref, g_ref, b_ref, w1_ref, b1_ref,
                  hn_ref, a_ref):
    hn = _ln(h_ref[...], g_ref[...], b_ref[...])
    hn_ref[...] = hn
    w1 = w1_ref[...]
    z = jnp.dot(hn, w1[:ENV_DIM]) + jnp.dot(hist_ref[...], w1[ENV_DIM:])
    a_ref[...] = jax.nn.relu(z + b1_ref[...])


def _build_a(h, hist, ln_g, ln_b, Wa1, b1):
    return pl.pallas_call(
        _build_a_body,
        grid=(_NB,),
        in_specs=[_row_spec(ENV_DIM), _row_spec(HIST_DIM),
                  _full_spec((1, ENV_DIM)), _full_spec((1, ENV_DIM)),
                  _full_spec((ENV_DIM + HIST_DIM, 2 * HIDDEN)),
                  _full_spec((1, 2 * HIDDEN))],
        out_specs=[_row_spec(ENV_DIM), _row_spec(_DT)],
        out_shape=[jax.ShapeDtypeStruct((N, ENV_DIM), jnp.float32),
                   jax.ShapeDtypeStruct((_NROW, _DT), jnp.float32)],
    )(h, hist, ln_g.reshape(1, -1), ln_b.reshape(1, -1), Wa1,
      b1.reshape(1, -1))


def _keep(agg, W2, b2, un):
    logits = jnp.dot(agg, W2) + b2
    g = -jnp.log(-jnp.log(un))
    t = (logits + g) / TAU
    m = jnp.max(t, axis=-1, keepdims=True)
    e = jnp.exp(t - m)
    y = e / jnp.sum(e, axis=-1, keepdims=True)
    return jnp.where(y[:, 0:1] >= y[:, 1:2], 1.0, 0.0)


def _act_post_body(p0_ref, p1_ref, d0_ref, d1_ref, uni_ref, uno_ref,
                   wi2_ref, bi2_ref, wo2_ref, bo2_ref, hn_ref, wc_ref,
                   bc_ref, c_ref, act_ref, ko_ref):
    p = p0_ref[...] + p1_ref[...]
    deg = jnp.maximum(d0_ref[...] + d1_ref[...], 1.0)
    ki = _keep(p[:, :HIDDEN] / deg, wi2_ref[...], bi2_ref[...], uni_ref[...])
    ko = _keep(p[:, HIDDEN:ENV_DIM] / deg, wo2_ref[...], bo2_ref[...],
               uno_ref[...])
    m = jnp.dot(hn_ref[...], wc_ref[...]) + bc_ref[...]
    c_ref[...] = ko * m
    ko_ref[...] = ko
    act_ref[...] = jnp.concatenate(
        [ki * ko, ki * (1.0 - ko), (1.0 - ki) * ko, (1.0 - ki) * (1.0 - ko)],
        axis=1)


def _act_post(p0, p1, d0, d1, un_in, un_out, Wa_in2, ba_in2, Wa_out2,
              ba_out2, hn, Wc, bc):
    return pl.pallas_call(
        _act_post_body,
        grid=(_NB,),
        in_specs=[_row_spec(_DT), _row_spec(_DT), _row_spec(1), _row_spec(1),
                  _row_spec(2), _row_spec(2),
                  _full_spec((HIDDEN, 2)), _full_spec((1, 2)),
                  _full_spec((HIDDEN, 2)), _full_spec((1, 2)),
                  _row_spec(ENV_DIM), _full_spec((ENV_DIM, ENV_DIM)),
                  _full_spec((1, ENV_DIM))],
        out_specs=[_row_spec(_DT), _row_spec(4), _row_spec(1)],
        out_shape=[jax.ShapeDtypeStruct((_NROW, _DT), jnp.float32),
                   jax.ShapeDtypeStruct((N, 4), jnp.float32),
                   jax.ShapeDtypeStruct((_NROW, 1), jnp.float32)],
    )(p0, p1, d0, d1, un_in, un_out, Wa_in2, ba_in2.reshape(1, -1), Wa_out2,
      ba_out2.reshape(1, -1), hn, Wc, bc.reshape(1, -1))


def _conv_post_body(s0_ref, s1_ref, w0_ref, w1_ref, hn_ref, act_ref, h_ref):
    s = s0_ref[...] + s1_ref[...]
    wdeg = jnp.maximum(w0_ref[...] + w1_ref[...], 1e-6)
    ki = act_ref[:, 0:1] + act_ref[:, 1:2]
    h_ref[...] = hn_ref[...] + ki * jax.nn.relu(s / wdeg)


def _conv_post(s0, s1, w0, w1, hn, act):
    return pl.pallas_call(
        _conv_post_body,
        grid=(_NB,),
        in_specs=[_row_spec(_DT), _row_spec(_DT), _row_spec(1), _row_spec(1),
                  _row_spec(ENV_DIM), _row_spec(4)],
        out_specs=_row_spec(ENV_DIM),
        out_shape=jax.ShapeDtypeStruct((N, ENV_DIM), jnp.float32),
    )(s0, s1, w0, w1, hn, act)


def _final_body(h_ref, g_ref, b_ref, batch_ref, wf_ref, bf_ref,
                pool_ref, res_ref):
    i = pl.program_id(0)

    @pl.when(i == 0)
    def _():
        pool_ref[...] = jnp.zeros((NUM_GRAPHS, ENV_DIM + 16), jnp.float32)

    rows = i * _R + lax.broadcasted_iota(jnp.int32, (_R, 1), 0)
    valid = rows < N
    hf = _ln(h_ref[...], g_ref[...], b_ref[...])
    hf = jnp.where(valid, hf, 0.0)
    gids = lax.broadcasted_iota(jnp.int32, (_R, NUM_GRAPHS), 1)
    onehot = jnp.where((batch_ref[...] == gids) & valid, 1.0, 0.0)
    hext = jnp.concatenate(
        [hf, valid.astype(jnp.float32), jnp.zeros((_R, 15), jnp.float32)],
        axis=1)
    pool_ref[...] += lax.dot_general(onehot, hext, (((0,), (0,)), ((), ())))

    @pl.when(i == _NB - 1)
    def _():
        p = pool_ref[...]
        pooled = p[:, :ENV_DIM] / jnp.maximum(p[:, ENV_DIM:ENV_DIM + 1], 1.0)
        res_ref[...] = jnp.dot(pooled, wf_ref[...]) + bf_ref[...]


def _final(h, ln_g, ln_b, batch2d, W_fin, b_fin):
    pooled, res = pl.pallas_call(
        _final_body,
        grid=(_NB,),
        in_specs=[_row_spec(ENV_DIM), _full_spec((1, ENV_DIM)),
                  _full_spec((1, ENV_DIM)), _row_spec(1),
                  _full_spec((ENV_DIM, OUT_DIM)), _full_spec((1, OUT_DIM))],
        out_specs=[_full_spec((NUM_GRAPHS, ENV_DIM + 16)),
                   _full_spec((NUM_GRAPHS, OUT_DIM))],
        out_shape=[jax.ShapeDtypeStruct((NUM_GRAPHS, ENV_DIM + 16),
                                        jnp.float32),
                   jax.ShapeDtypeStruct((NUM_GRAPHS, OUT_DIM), jnp.float32)],
    )(h, ln_g.reshape(1, -1), ln_b.reshape(1, -1), batch2d, W_fin,
      b_fin.reshape(1, -1))
    return res


def kernel(x, edge_index, pestat, batch, W_in, b_in, ln_g, ln_b, W_conv,
           b_conv, Wa_in1, ba_in1, Wa_in2, ba_in2, Wa_out1, ba_out1,
           Wa_out2, ba_out2, W_fin, b_fin):
    del pestat
    pad = jnp.full((_EPAD - E,), N, jnp.int32)
    u_p = jnp.concatenate([edge_index[0], pad])
    v_p = jnp.concatenate([edge_index[1], pad])
    Wa1 = jnp.concatenate([Wa_in1, Wa_out1], axis=1)
    b1 = jnp.concatenate([ba_in1, ba_out1])
    ones_k = jnp.ones((_NROW,), jnp.float32)

    uns = []
    for i in range(NUM_LAYERS):
        k = jax.random.fold_in(jax.random.key(42), i)
        uns.append(tuple(
            jax.random.uniform(jax.random.fold_in(k, j), (N, 2),
                               minval=1e-6, maxval=1.0 - 1e-6)
            for j in range(2)))

    h = _enc(x, W_in, b_in)
    hist = jnp.zeros((N, HIST_DIM), jnp.float32)
    acts = []
    for i in range(NUM_LAYERS):
        hn, A = _build_a(h, hist, ln_g, ln_b, Wa1, b1)
        pz, pd = _sc_pass(A, u_p, v_p, ones_k)
        ctab, act, ko = _act_post(
            pz[0], pz[1], pd[0].reshape(_NROW, 1), pd[1].reshape(_NROW, 1),
            uns[i][0], uns[i][1], Wa_in2, ba_in2, Wa_out2, ba_out2, hn,
            W_conv[i], b_conv[i])
        s, sw = _sc_pass(ctab, u_p, v_p, ko.reshape(_NROW))
        h = _conv_post(s[0], s[1], sw[0].reshape(_NROW, 1),
                       sw[1].reshape(_NROW, 1), hn, act)
        acts.append(act)
        if i < NUM_LAYERS - 1:
            hist = jnp.concatenate([hist[:, 4:], act], axis=1)

    result = _final(h, ln_g, ln_b, batch.reshape(N, 1), W_fin, b_fin)
    history = jnp.concatenate(
        [jnp.zeros((N, 4), x.dtype), acts[0], acts[1]], axis=1)
    return (result, -jnp.ones((NUM_LAYERS,), x.dtype), history)


# speedup vs baseline: 12.4548x; 1.4268x over previous
import functools

import jax
import jax.numpy as jnp
from jax import lax
from jax.experimental import pallas as pl
from jax.experimental.pallas import tpu as pltpu
from jax.experimental.pallas import tpu_sc as plsc

N = 10000
E = 320000
D_FEAT = 128
ENV_DIM = 128
HIDDEN = 64
NUM_LAYERS = 3
HIST_DIM = 12
NUM_GRAPHS = 64
OUT_DIM = 40
TAU = 0.5

_NC, _NS = 2, 16
_NW = _NC * _NS
_C = 128
_NCHUNK = 79
_EPW = _C * _NCHUNK
_EPAD = _EPW * _NW
_DT = 128
_NROW = 10240
_RPT = _NROW // _NS
_HR = _NROW // 128

_R = 512
_NB = _NROW // _R


def _sc_pass_body(table, u, v, kvec, out, out_h,
                  u_idx, v_idx, rows, kv_v, hist, idx_h, acc, acc_h, sem):
    cid = lax.axis_index("c")
    sid = lax.axis_index("s")
    wid = cid * _NS + sid

    zero16 = jnp.zeros((16,), jnp.float32)

    def _zr(i, carry):
        for j in range(_DT // 16):
            rows[i, pl.ds(j * 16, 16)] = zero16
        return carry
    lax.fori_loop(0, _C, _zr, 0)

    def _zh(i, carry):
        for j in range(128 // 16):
            hist[i, pl.ds(j * 16, 16)] = zero16
        return carry
    lax.fori_loop(0, _HR, _zh, 0)

    iota16 = lax.iota(jnp.int32, 16)
    for j in range(_HR // 16):
        idx_h[pl.ds(j * 16, 16)] = iota16 + j * 16

    pltpu.sync_copy(kvec, kv_v)

    for k in range(_RPT // _C):
        pltpu.sync_copy(rows, acc.at[pl.ds(sid * _RPT + k * _C, _C)])

    @pl.when(sid < _HR // 8)
    def _():
        pltpu.sync_copy(rows.at[pl.ds(0, 8)], acc_h.at[pl.ds(sid * 8, 8)])
    plsc.subcore_barrier()

    base = wid * _EPW

    def _chunk(ci, carry):
        off = base + ci * _C
        pltpu.sync_copy(u.at[pl.ds(off, _C)], u_idx)
        pltpu.sync_copy(v.at[pl.ds(off, _C)], v_idx)
        gather = pltpu.async_copy(table.at[u_idx], rows, sem)
        for j in range(_C // 16):
            u16 = u_idx[pl.ds(j * 16, 16)]
            v16 = v_idx[pl.ds(j * 16, 16)]
            kv16 = plsc.load_gather(kv_v, [u16])
            plsc.addupdate_scatter(hist, [v16 >> 7, v16 & 127], kv16)
        gather.wait()
        pltpu.sync_copy(rows, acc.at[v_idx], add=True)
        return carry
    lax.fori_loop(0, _NCHUNK, _chunk, 0)

    pltpu.sync_copy(hist, acc_h.at[idx_h], add=True)
    plsc.subcore_barrier()

    pltpu.sync_copy(acc.at[pl.ds(sid * _RPT, _RPT)],
                    out.at[cid, pl.ds(sid * _RPT, _RPT)])

    @pl.when(sid < _HR // 8)
    def _():
        pltpu.sync_copy(acc_h.at[pl.ds(sid * 8, 8)],
                        out_h.at[cid, pl.ds(sid * 8, 8)])


@functools.cache
def _get_sc_pass():
    return pl.kernel(
        _sc_pass_body,
        out_type=(jax.ShapeDtypeStruct((_NC, _NROW, _DT), jnp.float32),
                  jax.ShapeDtypeStruct((_NC, _HR, 128), jnp.float32)),
        mesh=plsc.VectorSubcoreMesh(core_axis_name="c", subcore_axis_name="s",
                                    num_cores=_NC, num_subcores=_NS),
        compiler_params=pltpu.CompilerParams(needs_layout_passes=False),
        scratch_types=[
            pltpu.VMEM((_C,), jnp.int32),
            pltpu.VMEM((_C,), jnp.int32),
            pltpu.VMEM((_C, _DT), jnp.float32),
            pltpu.VMEM((_NROW,), jnp.float32),
            pltpu.VMEM((_HR, 128), jnp.float32),
            pltpu.VMEM((_HR,), jnp.int32),
            pltpu.VMEM_SHARED((_NROW, _DT), jnp.float32),
            pltpu.VMEM_SHARED((_HR, 128), jnp.float32),
            pltpu.SemaphoreType.DMA,
        ],
    )


def _sc_pass(table, u, v, kvec):
    return _get_sc_pass()(table, u, v, kvec)


_ZL = _NROW * 4
_AR = (_ZL + _NROW) // 128


def _sc_act_body(zz, u, v, out, zz_v, u_all, v_all, idx80, acc_v, acc_sh):
    cid = lax.axis_index("c")
    sid = lax.axis_index("s")
    wid = cid * _NS + sid

    zero16 = jnp.zeros((16,), jnp.float32)

    def _za(i, carry):
        for j in range(128 // 16):
            acc_v[i, pl.ds(j * 16, 16)] = zero16
        return carry
    lax.fori_loop(0, _AR, _za, 0)

    @pl.when(sid < 10)
    def _():
        pltpu.sync_copy(acc_v.at[pl.ds(0, _AR // 10)],
                        acc_sh.at[pl.ds(sid * (_AR // 10), _AR // 10)])

    pltpu.sync_copy(zz, zz_v)
    pltpu.sync_copy(u.at[pl.ds(wid * _EPW, _EPW)], u_all)
    pltpu.sync_copy(v.at[pl.ds(wid * _EPW, _EPW)], v_all)
    plsc.subcore_barrier()

    ones16 = jnp.ones((16,), jnp.float32)

    def _body(i, carry):
        u16 = u_all[pl.ds(i * 16, 16)]
        v16 = v_all[pl.ds(i * 16, 16)]
        ub = u16 << 2
        vb = v16 << 2
        for ch in range(4):
            g16 = plsc.load_gather(zz_v, [ub + ch])
            fc = vb + ch
            plsc.addupdate_scatter(acc_v, [fc >> 7, fc & 127], g16)
        plsc.addupdate_scatter(acc_v, [(v16 >> 7) + (_ZL // 128), v16 & 127],
                               ones16)
        return carry
    lax.fori_loop(0, _EPW // 16, _body, 0)

    iota16 = lax.iota(jnp.int32, 16)
    for j in range(_AR // 80):
        for jj in range(80 // 16):
            idx80[pl.ds(jj * 16, 16)] = iota16 + (j * 80 + jj * 16)
        pltpu.sync_copy(acc_v.at[pl.ds(j * 80, 80)], acc_sh.at[idx80],
                        add=True)
    plsc.subcore_barrier()

    @pl.when(sid < 10)
    def _():
        pltpu.sync_copy(acc_sh.at[pl.ds(sid * (_AR // 10), _AR // 10)],
                        out.at[cid, pl.ds(sid * (_AR // 10), _AR // 10)])


@functools.cache
def _get_sc_act():
    return pl.kernel(
        _sc_act_body,
        out_type=jax.ShapeDtypeStruct((_NC, _AR, 128), jnp.float32),
        mesh=plsc.VectorSubcoreMesh(core_axis_name="c", subcore_axis_name="s",
                                    num_cores=_NC, num_subcores=_NS),
        compiler_params=pltpu.CompilerParams(needs_layout_passes=False),
        scratch_types=[
            pltpu.VMEM((_ZL,), jnp.float32),
            pltpu.VMEM((_EPW,), jnp.int32),
            pltpu.VMEM((_EPW,), jnp.int32),
            pltpu.VMEM((80,), jnp.int32),
            pltpu.VMEM((_AR, 128), jnp.float32),
            pltpu.VMEM_SHARED((_AR, 128), jnp.float32),
        ],
    )


def _sc_act(zz_flat, u, v):
    return _get_sc_act()(zz_flat, u, v)


def _row_spec(w):
    return pl.BlockSpec((_R, w), lambda i: (i, 0))


def _full_spec(shape):
    nd = len(shape)
    return pl.BlockSpec(shape, lambda i, _n=nd: (0,) * _n)


def _ln(hb, g, b):
    mu = jnp.mean(hb, axis=-1, keepdims=True)
    var = jnp.mean((hb - mu) ** 2, axis=-1, keepdims=True)
    return (hb - mu) / jnp.sqrt(var + 1e-5) * g + b


def _enc_body(x_ref, w_ref, b_ref, h_ref):
    h_ref[...] = jax.nn.relu(jnp.dot(x_ref[...], w_ref[...]) + b_ref[...])


def _enc(x, W_in, b_in):
    return pl.pallas_call(
        _enc_body,
        grid=(_NB,),
        in_specs=[_row_spec(D_FEAT), _full_spec((D_FEAT, ENV_DIM)),
                  _full_spec((1, ENV_DIM))],
        out_specs=_row_spec(ENV_DIM),
        out_shape=jax.ShapeDtypeStruct((N, ENV_DIM), jnp.float32),
    )(x, W_in, b_in.reshape(1, ENV_DIM))


def _build_a_body(h_ref, hist_ref, g_ref, b_ref, w1_ref, b1_ref, w2_ref,
                  hn_ref, za_ref):
    hn = _ln(h_ref[...], g_ref[...], b_ref[...])
    hn_ref[...] = hn
    w1 = w1_ref[...]
    z = jnp.dot(hn, w1[:ENV_DIM]) + jnp.dot(hist_ref[...], w1[ENV_DIM:])
    z = jax.nn.relu(z + b1_ref[...])
    za_ref[...] = jnp.dot(z, w2_ref[...])


def _build_a(h, hist, ln_g, ln_b, Wa1, b1, W2blk):
    return pl.pallas_call(
        _build_a_body,
        grid=(_NB,),
        in_specs=[_row_spec(ENV_DIM), _row_spec(HIST_DIM),
                  _full_spec((1, ENV_DIM)), _full_spec((1, ENV_DIM)),
                  _full_spec((ENV_DIM + HIST_DIM, 2 * HIDDEN)),
                  _full_spec((1, 2 * HIDDEN)),
                  _full_spec((2 * HIDDEN, 4))],
        out_specs=[_row_spec(ENV_DIM), _row_spec(4)],
        out_shape=[jax.ShapeDtypeStruct((N, ENV_DIM), jnp.float32),
                   jax.ShapeDtypeStruct((_NROW, 4), jnp.float32)],
    )(h, hist, ln_g.reshape(1, -1), ln_b.reshape(1, -1), Wa1,
      b1.reshape(1, -1), W2blk)


def _keep(logits, un):
    g = -jnp.log(-jnp.log(un))
    t = (logits + g) / TAU
    m = jnp.max(t, axis=-1, keepdims=True)
    e = jnp.exp(t - m)
    y = e / jnp.sum(e, axis=-1, keepdims=True)
    return jnp.where(y[:, 0:1] >= y[:, 1:2], 1.0, 0.0)


def _act_post_body(p0_ref, p1_ref, d0_ref, d1_ref, uni_ref, uno_ref,
                   bi2_ref, bo2_ref, hn_ref, wc_ref,
                   bc_ref, c_ref, act_ref, ko_ref):
    p = p0_ref[...] + p1_ref[...]
    deg = jnp.maximum(d0_ref[...] + d1_ref[...], 1.0)
    ki = _keep(p[:, 0:2] / deg + bi2_ref[...], uni_ref[...])
    ko = _keep(p[:, 2:4] / deg + bo2_ref[...], uno_ref[...])
    m = jnp.dot(hn_ref[...], wc_ref[...]) + bc_ref[...]
    c_ref[...] = ko * m
    ko_ref[...] = ko
    act_ref[...] = jnp.concatenate(
        [ki * ko, ki * (1.0 - ko), (1.0 - ki) * ko, (1.0 - ki) * (1.0 - ko)],
        axis=1)


def _act_post(p0, p1, d0, d1, un_in, un_out, ba_in2, ba_out2, hn, Wc, bc):
    return pl.pallas_call(
        _act_post_body,
        grid=(_NB,),
        in_specs=[_row_spec(4), _row_spec(4), _row_spec(1), _row_spec(1),
                  _row_spec(2), _row_spec(2),
                  _full_spec((1, 2)), _full_spec((1, 2)),
                  _row_spec(ENV_DIM), _full_spec((ENV_DIM, ENV_DIM)),
                  _full_spec((1, ENV_DIM))],
        out_specs=[_row_spec(_DT), _row_spec(4), _row_spec(1)],
        out_shape=[jax.ShapeDtypeStruct((_NROW, _DT), jnp.float32),
                   jax.ShapeDtypeStruct((N, 4), jnp.float32),
                   jax.ShapeDtypeStruct((_NROW, 1), jnp.float32)],
    )(p0, p1, d0, d1, un_in, un_out, ba_in2.reshape(1, -1),
      ba_out2.reshape(1, -1), hn, Wc, bc.reshape(1, -1))


def _conv_post_body(s0_ref, s1_ref, w0_ref, w1_ref, hn_ref, act_ref, h_ref):
    s = s0_ref[...] + s1_ref[...]
    wdeg = jnp.maximum(w0_ref[...] + w1_ref[...], 1e-6)
    ki = act_ref[:, 0:1] + act_ref[:, 1:2]
    h_ref[...] = hn_ref[...] + ki * jax.nn.relu(s / wdeg)


def _conv_post(s0, s1, w0, w1, hn, act):
    return pl.pallas_call(
        _conv_post_body,
        grid=(_NB,),
        in_specs=[_row_spec(_DT), _row_spec(_DT), _row_spec(1), _row_spec(1),
                  _row_spec(ENV_DIM), _row_spec(4)],
        out_specs=_row_spec(ENV_DIM),
        out_shape=jax.ShapeDtypeStruct((N, ENV_DIM), jnp.float32),
    )(s0, s1, w0, w1, hn, act)


def _final_body(h_ref, g_ref, b_ref, batch_ref, wf_ref, bf_ref,
                pool_ref, res_ref):
    i = pl.program_id(0)

    @pl.when(i == 0)
    def _():
        pool_ref[...] = jnp.zeros((NUM_GRAPHS, ENV_DIM + 16), jnp.float32)

    rows = i * _R + lax.broadcasted_iota(jnp.int32, (_R, 1), 0)
    valid = rows < N
    hf = _ln(h_ref[...], g_ref[...], b_ref[...])
    hf = jnp.where(valid, hf, 0.0)
    gids = lax.broadcasted_iota(jnp.int32, (_R, NUM_GRAPHS), 1)
    onehot = jnp.where((batch_ref[...] == gids) & valid, 1.0, 0.0)
    hext = jnp.concatenate(
        [hf, valid.astype(jnp.float32), jnp.zeros((_R, 15), jnp.float32)],
        axis=1)
    pool_ref[...] += lax.dot_general(onehot, hext, (((0,), (0,)), ((), ())))

    @pl.when(i == _NB - 1)
    def _():
        p = pool_ref[...]
        pooled = p[:, :ENV_DIM] / jnp.maximum(p[:, ENV_DIM:ENV_DIM + 1], 1.0)
        res_ref[...] = jnp.dot(pooled, wf_ref[...]) + bf_ref[...]


def _final(h, ln_g, ln_b, batch2d, W_fin, b_fin):
    pooled, res = pl.pallas_call(
        _final_body,
        grid=(_NB,),
        in_specs=[_row_spec(ENV_DIM), _full_spec((1, ENV_DIM)),
                  _full_spec((1, ENV_DIM)), _row_spec(1),
                  _full_spec((ENV_DIM, OUT_DIM)), _full_spec((1, OUT_DIM))],
        out_specs=[_full_spec((NUM_GRAPHS, ENV_DIM + 16)),
                   _full_spec((NUM_GRAPHS, OUT_DIM))],
        out_shape=[jax.ShapeDtypeStruct((NUM_GRAPHS, ENV_DIM + 16),
                                        jnp.float32),
                   jax.ShapeDtypeStruct((NUM_GRAPHS, OUT_DIM), jnp.float32)],
    )(h, ln_g.reshape(1, -1), ln_b.reshape(1, -1), batch2d, W_fin,
      b_fin.reshape(1, -1))
    return res


def kernel(x, edge_index, pestat, batch, W_in, b_in, ln_g, ln_b, W_conv,
           b_conv, Wa_in1, ba_in1, Wa_in2, ba_in2, Wa_out1, ba_out1,
           Wa_out2, ba_out2, W_fin, b_fin):
    del pestat
    pad = jnp.full((_EPAD - E,), N, jnp.int32)
    u_p = jnp.concatenate([edge_index[0], pad])
    v_p = jnp.concatenate([edge_index[1], pad])
    Wa1 = jnp.concatenate([Wa_in1, Wa_out1], axis=1)
    b1 = jnp.concatenate([ba_in1, ba_out1])
    z2 = jnp.zeros((HIDDEN, 2), jnp.float32)
    W2blk = jnp.concatenate(
        [jnp.concatenate([Wa_in2, z2], axis=1),
         jnp.concatenate([z2, Wa_out2], axis=1)], axis=0)

    uns = []
    for i in range(NUM_LAYERS):
        k = jax.random.fold_in(jax.random.key(42), i)
        uns.append(tuple(
            jax.random.uniform(jax.random.fold_in(k, j), (N, 2),
                               minval=1e-6, maxval=1.0 - 1e-6)
            for j in range(2)))

    h = _enc(x, W_in, b_in)
    hist = jnp.zeros((N, HIST_DIM), jnp.float32)
    acts = []
    for i in range(NUM_LAYERS):
        hn, za = _build_a(h, hist, ln_g, ln_b, Wa1, b1, W2blk)
        pact = _sc_act(za.reshape(_ZL), u_p, v_p)
        ctab, act, ko = _act_post(
            pact[0, :_ZL // 128].reshape(_NROW, 4),
            pact[1, :_ZL // 128].reshape(_NROW, 4),
            pact[0, _ZL // 128:].reshape(_NROW, 1),
            pact[1, _ZL // 128:].reshape(_NROW, 1),
            uns[i][0], uns[i][1], ba_in2, ba_out2, hn,
            W_conv[i], b_conv[i])
        s, sw = _sc_pass(ctab, u_p, v_p, ko.reshape(_NROW))
        h = _conv_post(s[0], s[1], sw[0].reshape(_NROW, 1),
                       sw[1].reshape(_NROW, 1), hn, act)
        acts.append(act)
        if i < NUM_LAYERS - 1:
            hist = jnp.concatenate([hist[:, 4:], act], axis=1)

    result = _final(h, ln_g, ln_b, batch.reshape(N, 1), W_fin, b_fin)
    history = jnp.concatenate(
        [jnp.zeros((N, 4), x.dtype), acts[0], acts[1]], axis=1)
    return (result, -jnp.ones((NUM_LAYERS,), x.dtype), history)
